# Initial kernel scaffold; baseline (speedup 1.0000x reference)
import jax
import jax.numpy as jnp
from jax.experimental import pallas as pl


def kernel(x, edge_index, emb, W1, b1, W2, b2, W3, b3, W4, b4):
    raise NotImplementedError("write your pallas kernel here")


# speedup vs baseline: 8.2776x; 8.2776x over previous
import functools

import jax
import jax.numpy as jnp
from jax import lax
from jax.experimental import pallas as pl
from jax.experimental.pallas import tpu as pltpu
from jax.experimental.pallas import tpu_sc as plsc

_N = 10000
_E = 160000
_D = 512
_NP = 10240
_EP = 163840
_NW = 32
_NCH = 40
_RPT = _NP // 16
_BN = 512
_NCK = 4


def _fill(buf, n, val, dtype):
    v = jnp.full((16,), val, dtype)

    @pl.loop(0, n // 16)
    def _(i):
        buf[pl.ds(i * 16, 16)] = v


def _sc_deg_body(dst_hbm, degp_hbm, acc, dstbuf, ones, zbuf):
    c = lax.axis_index("c")
    s = lax.axis_index("s")
    wid = c * 16 + s
    _fill(zbuf, _RPT, 0.0, jnp.float32)
    _fill(ones, 128, 1.0, jnp.float32)
    pltpu.sync_copy(zbuf, acc.at[pl.ds(s * _RPT, _RPT)])
    pltpu.sync_copy(dst_hbm.at[wid], dstbuf)
    plsc.subcore_barrier()

    @pl.loop(0, _NCH)
    def _(j):
        pltpu.sync_copy(ones, acc.at[dstbuf.at[j]], add=True)

    plsc.subcore_barrier()
    sl = pl.ds(s * _RPT, _RPT)
    pltpu.sync_copy(acc.at[sl], degp_hbm.at[c, sl])


def _sc_cmat_body(src_hbm, dst_hbm, x_hbm, dinv_hbm, cp_hbm,
                  acc, srcbuf, dstbuf, xbuf, dinvbuf, valbuf, fidx, zbuf):
    c = lax.axis_index("c")
    s = lax.axis_index("s")
    wid = c * 16 + s
    _fill(zbuf, _RPT, 0.0, jnp.float32)

    @pl.loop(0, 8)
    def _(t):
        pltpu.sync_copy(zbuf, acc.at[pl.ds((s * 8 + t) * _RPT, _RPT)])

    pltpu.sync_copy(src_hbm.at[wid], srcbuf)
    pltpu.sync_copy(dst_hbm.at[wid], dstbuf)
    pltpu.sync_copy(x_hbm, xbuf)
    pltpu.sync_copy(dinv_hbm, dinvbuf)
    plsc.subcore_barrier()

    @pl.loop(0, _NCH)
    def _(j):
        for k in range(8):
            sl = pl.ds(k * 16, 16)
            s16 = srcbuf[j, sl]
            d16 = dstbuf[j, sl]
            xs = plsc.load_gather(xbuf, [s16])
            dv = plsc.load_gather(dinvbuf, [s16])
            valbuf[0, sl] = dv
            fidx[0, sl] = d16 * 8 + xs
        pltpu.sync_copy(valbuf.at[0], acc.at[fidx.at[0]], add=True)

    plsc.subcore_barrier()

    @pl.loop(0, 8)
    def _(t):
        sl = pl.ds((s * 8 + t) * _RPT, _RPT)
        pltpu.sync_copy(acc.at[sl], cp_hbm.at[c, sl])


def _sc_uprop_body(src_hbm, dst_hbm, u_hbm, vp_hbm,
                   acc, srcbuf, dstbuf, ubuf, gbuf, zbuf):
    c = lax.axis_index("c")
    s = lax.axis_index("s")
    wid = c * 16 + s
    _fill(zbuf, _RPT, 0.0, jnp.float32)
    pltpu.sync_copy(zbuf, acc.at[pl.ds(s * _RPT, _RPT)])
    pltpu.sync_copy(src_hbm.at[wid], srcbuf)
    pltpu.sync_copy(dst_hbm.at[wid], dstbuf)
    pltpu.sync_copy(u_hbm, ubuf)
    plsc.subcore_barrier()

    @pl.loop(0, _NCH)
    def _(j):
        for k in range(8):
            sl = pl.ds(k * 16, 16)
            s16 = srcbuf[j, sl]
            gbuf[0, sl] = plsc.load_gather(ubuf, [s16])
        pltpu.sync_copy(gbuf.at[0], acc.at[dstbuf.at[j]], add=True)

    plsc.subcore_barrier()
    sl = pl.ds(s * _RPT, _RPT)
    pltpu.sync_copy(acc.at[sl], vp_hbm.at[c, sl])


def _sc_prop_body(qf_hbm, src_hbm, dst_hbm, sf_hbm,
                  acc, srcbuf, dstbuf, r0, r1, gs0, gs1, ss0, ss1):
    c = lax.axis_index("c")
    s = lax.axis_index("s")

    for r in range(2):
        chunk = 2 * r + c
        base = chunk * _NP

        pltpu.sync_copy(qf_hbm.at[pl.ds(base + s * _RPT, _RPT)],
                        acc.at[pl.ds(s * _RPT, _RPT)])
        plsc.subcore_barrier()

        for w in range(2):
            wid = s + 16 * w
            pltpu.sync_copy(src_hbm.at[wid], srcbuf)
            pltpu.sync_copy(dst_hbm.at[wid], dstbuf)

            @pl.loop(0, _NCH)
            def _(j):
                for k in range(8):
                    sl = pl.ds(k * 16, 16)
                    srcbuf[j, sl] = srcbuf[j, sl] + base

            @pl.loop(0, _NCH, step=2)
            def _(j):
                g0 = pltpu.async_copy(qf_hbm.at[srcbuf.at[j]], r0, gs0)
                g1 = pltpu.async_copy(qf_hbm.at[srcbuf.at[j + 1]], r1, gs1)
                g0.wait()
                s0 = pltpu.async_copy(r0, acc.at[dstbuf.at[j]], ss0,
                                      add=True)
                g1.wait()
                s1 = pltpu.async_copy(r1, acc.at[dstbuf.at[j + 1]], ss1,
                                      add=True)
                s0.wait()
                s1.wait()

        plsc.subcore_barrier()
        pltpu.sync_copy(acc.at[pl.ds(s * _RPT, _RPT)],
                        sf_hbm.at[pl.ds(base + s * _RPT, _RPT)])
        plsc.subcore_barrier()


def _tc_prep_body(degp_ref, dinv_ref):
    p = degp_ref[0] + degp_ref[1]
    dinv_ref[...] = lax.rsqrt(1.0 + p)


def _tc_l1_body(cp_ref, x_ref, dinv_ref, emb_ref, w1_ref, b1_ref, w2_ref,
                q_ref):
    f32 = jnp.float32
    cmat = cp_ref[0] + cp_ref[1]
    xi = x_ref[...][:, 0]
    dv = dinv_ref[...]
    classes = lax.broadcasted_iota(jnp.int32, (_BN, 8), 1)
    cmat = cmat + jnp.where(classes == xi[:, None], dv, 0.0)
    z = lax.dot_general(emb_ref[...], w1_ref[...],
                        (((1,), (1,)), ((), ())),
                        preferred_element_type=f32)
    s1 = lax.dot_general(cmat, z, (((1,), (0,)), ((), ())),
                         preferred_element_type=f32)
    h = jnp.maximum(dv * s1 + b1_ref[...], 0.0)
    q = dv * lax.dot_general(h, w2_ref[...], (((1,), (1,)), ((), ())),
                             preferred_element_type=f32)
    for ck in range(_NCK):
        q_ref[ck] = q[:, ck * 128:(ck + 1) * 128]


def _tc_mid_body(s_ref, dinv_ref, b_ref, w_ref, q_ref):
    f32 = jnp.float32
    sfull = jnp.concatenate([s_ref[ck] for ck in range(_NCK)], axis=-1)
    dv = dinv_ref[...]
    h = jnp.maximum(dv * sfull + b_ref[...], 0.0)
    q = dv * lax.dot_general(h, w_ref[...], (((1,), (1,)), ((), ())),
                             preferred_element_type=f32)
    for ck in range(_NCK):
        q_ref[ck] = q[:, ck * 128:(ck + 1) * 128]


def _tc_head_body(s_ref, dinv_ref, b_ref, w4_ref, u_ref):
    f32 = jnp.float32
    sfull = jnp.concatenate([s_ref[ck] for ck in range(_NCK)], axis=-1)
    dv = dinv_ref[...]
    h = jnp.maximum(dv * sfull + b_ref[...], 0.0)
    t = lax.dot_general(h, w4_ref[...], (((1,), (1,)), ((), ())),
                        preferred_element_type=f32)
    u_ref[...] = dv * t


def _tc_fin_body(u_ref, vp_ref, dinv_ref, b4_ref, o_ref):
    v = u_ref[...] + vp_ref[0] + vp_ref[1]
    o_ref[...] = jax.nn.sigmoid(dinv_ref[...] * v + b4_ref[0, 0])


def _build(interpret=False):
    f32 = jnp.float32
    i32 = jnp.int32
    mesh = plsc.VectorSubcoreMesh(core_axis_name="c", subcore_axis_name="s",
                                  num_cores=2, num_subcores=16)
    sc_params = pltpu.CompilerParams(needs_layout_passes=False)

    sc_deg = pl.kernel(
        _sc_deg_body,
        out_type=jax.ShapeDtypeStruct((2, _NP), f32),
        mesh=mesh,
        scratch_types=[
            pltpu.VMEM_SHARED((_NP,), f32),
            pltpu.VMEM((_NCH, 128), i32),
            pltpu.VMEM((128,), f32),
            pltpu.VMEM((_RPT,), f32),
        ],
        interpret=interpret,
        compiler_params=sc_params,
    )

    sc_cmat = pl.kernel(
        _sc_cmat_body,
        out_type=jax.ShapeDtypeStruct((2, 8 * _NP), f32),
        mesh=mesh,
        scratch_types=[
            pltpu.VMEM_SHARED((8 * _NP,), f32),
            pltpu.VMEM((_NCH, 128), i32),
            pltpu.VMEM((_NCH, 128), i32),
            pltpu.VMEM((_NP,), i32),
            pltpu.VMEM((_NP,), f32),
            pltpu.VMEM((1, 128), f32),
            pltpu.VMEM((1, 128), i32),
            pltpu.VMEM((_RPT,), f32),
        ],
        interpret=interpret,
        compiler_params=sc_params,
    )

    sc_uprop = pl.kernel(
        _sc_uprop_body,
        out_type=jax.ShapeDtypeStruct((2, _NP), f32),
        mesh=mesh,
        scratch_types=[
            pltpu.VMEM_SHARED((_NP,), f32),
            pltpu.VMEM((_NCH, 128), i32),
            pltpu.VMEM((_NCH, 128), i32),
            pltpu.VMEM((_NP,), f32),
            pltpu.VMEM((1, 128), f32),
            pltpu.VMEM((_RPT,), f32),
        ],
        interpret=interpret,
        compiler_params=sc_params,
    )

    sc_prop = pl.kernel(
        _sc_prop_body,
        out_type=jax.ShapeDtypeStruct((_NCK * _NP, 128), f32),
        mesh=mesh,
        scratch_types=[
            pltpu.VMEM_SHARED((_NP, 128), f32),
            pltpu.VMEM((_NCH, 128), i32),
            pltpu.VMEM((_NCH, 128), i32),
            pltpu.VMEM((128, 128), f32),
            pltpu.VMEM((128, 128), f32),
            pltpu.SemaphoreType.DMA,
            pltpu.SemaphoreType.DMA,
            pltpu.SemaphoreType.DMA,
            pltpu.SemaphoreType.DMA,
        ],
        interpret=interpret,
        compiler_params=sc_params,
    )

    ngrid = _NP // _BN
    tc_prep = pl.pallas_call(
        _tc_prep_body,
        grid=(ngrid,),
        in_specs=[pl.BlockSpec((2, _BN, 1), lambda i: (0, i, 0))],
        out_specs=pl.BlockSpec((_BN, 1), lambda i: (i, 0)),
        out_shape=jax.ShapeDtypeStruct((_NP, 1), f32),
        interpret=interpret,
    )

    tc_l1 = pl.pallas_call(
        _tc_l1_body,
        grid=(ngrid,),
        in_specs=[
            pl.BlockSpec((2, _BN, 8), lambda i: (0, i, 0)),
            pl.BlockSpec((_BN, 1), lambda i: (i, 0)),
            pl.BlockSpec((_BN, 1), lambda i: (i, 0)),
            pl.BlockSpec((8, _D), lambda i: (0, 0)),
            pl.BlockSpec((_D, _D), lambda i: (0, 0)),
            pl.BlockSpec((1, _D), lambda i: (0, 0)),
            pl.BlockSpec((_D, _D), lambda i: (0, 0)),
        ],
        out_specs=pl.BlockSpec((_NCK, _BN, 128), lambda i: (0, i, 0)),
        out_shape=jax.ShapeDtypeStruct((_NCK, _NP, 128), f32),
        interpret=interpret,
    )

    tc_mid = pl.pallas_call(
        _tc_mid_body,
        grid=(ngrid,),
        in_specs=[
            pl.BlockSpec((_NCK, _BN, 128), lambda i: (0, i, 0)),
            pl.BlockSpec((_BN, 1), lambda i: (i, 0)),
            pl.BlockSpec((1, _D), lambda i: (0, 0)),
            pl.BlockSpec((_D, _D), lambda i: (0, 0)),
        ],
        out_specs=pl.BlockSpec((_NCK, _BN, 128), lambda i: (0, i, 0)),
        out_shape=jax.ShapeDtypeStruct((_NCK, _NP, 128), f32),
        interpret=interpret,
    )

    tc_head = pl.pallas_call(
        _tc_head_body,
        grid=(ngrid,),
        in_specs=[
            pl.BlockSpec((_NCK, _BN, 128), lambda i: (0, i, 0)),
            pl.BlockSpec((_BN, 1), lambda i: (i, 0)),
            pl.BlockSpec((1, _D), lambda i: (0, 0)),
            pl.BlockSpec((1, _D), lambda i: (0, 0)),
        ],
        out_specs=pl.BlockSpec((_BN, 1), lambda i: (i, 0)),
        out_shape=jax.ShapeDtypeStruct((_NP, 1), f32),
        interpret=interpret,
    )

    tc_fin = pl.pallas_call(
        _tc_fin_body,
        grid=(ngrid,),
        in_specs=[
            pl.BlockSpec((_BN, 1), lambda i: (i, 0)),
            pl.BlockSpec((2, _BN, 1), lambda i: (0, i, 0)),
            pl.BlockSpec((_BN, 1), lambda i: (i, 0)),
            pl.BlockSpec((1, 1), lambda i: (0, 0)),
        ],
        out_specs=pl.BlockSpec((_BN, 1), lambda i: (i, 0)),
        out_shape=jax.ShapeDtypeStruct((_NP, 1), f32),
        interpret=interpret,
    )

    return dict(sc_deg=sc_deg, sc_cmat=sc_cmat, sc_uprop=sc_uprop,
                sc_prop=sc_prop, tc_prep=tc_prep, tc_l1=tc_l1,
                tc_mid=tc_mid, tc_head=tc_head, tc_fin=tc_fin)


_K = _build()


def kernel(x, edge_index, emb, W1, b1, W2, b2, W3, b3, W4, b4):
    f32 = jnp.float32
    i32 = jnp.int32
    src = edge_index[0].astype(i32)
    dst = edge_index[1].astype(i32)
    pad_e = jnp.full((_EP - _E,), _N, i32)
    srcp = jnp.concatenate([src, pad_e]).reshape(_NW, _NCH, 128)
    dstp = jnp.concatenate([dst, pad_e]).reshape(_NW, _NCH, 128)
    xp = jnp.concatenate([x.astype(i32), jnp.zeros((_NP - _N,), i32)])
    emb8 = jnp.zeros((8, _D), f32).at[:3, :].set(emb)

    degp = _K["sc_deg"](dstp)
    dinv = _K["tc_prep"](degp.reshape(2, _NP, 1))
    cp = _K["sc_cmat"](srcp, dstp, xp, dinv.reshape(_NP))
    q = _K["tc_l1"](cp.reshape(2, _NP, 8), xp.reshape(_NP, 1), dinv,
                    emb8, W1, b1.reshape(1, _D), W2)
    sf = _K["sc_prop"](q.reshape(_NCK * _NP, 128), srcp, dstp)
    q = _K["tc_mid"](sf.reshape(_NCK, _NP, 128), dinv,
                     b2.reshape(1, _D), W3)
    sf = _K["sc_prop"](q.reshape(_NCK * _NP, 128), srcp, dstp)
    u = _K["tc_head"](sf.reshape(_NCK, _NP, 128), dinv,
                      b3.reshape(1, _D), W4)
    vp = _K["sc_uprop"](srcp, dstp, u.reshape(_NP))
    o = _K["tc_fin"](u, vp.reshape(2, _NP, 1), dinv,
                     b4.reshape(1, 1))
    return o.reshape(_NP)[:_N]
